# baseline (speedup 1.0000x reference)
import jax
import jax.numpy as jnp
from jax.experimental import pallas as pl
from jax.experimental.pallas import tpu as pltpu

_L, _B, _D, _T, _R, _C = 12, 1024, 768, 8, 16, 100
_TR = _T * _R


def _first_max_onehot(scores):
    t = scores.shape[-1]
    m = jnp.max(scores, axis=-1, keepdims=True)
    is_max = (scores == m).astype(jnp.float32)
    tri = (jax.lax.broadcasted_iota(jnp.int32, (t, t), 0)
           <= jax.lax.broadcasted_iota(jnp.int32, (t, t), 1)).astype(jnp.float32)
    csum = jax.lax.dot_general(is_max, tri, (((1,), (0,)), ((), ())),
                              precision=jax.lax.Precision.HIGHEST)
    return is_max * (csum == 1.0).astype(jnp.float32)


def _smile_kernel(x0_ref, w0_ref, v_ref, u_ref, hw_ref, hb_ref, out_ref,
                  x_s, cnt_s):
    l = pl.program_id(0)

    @pl.when(l == 0)
    def _init():
        x_s[...] = x0_ref[...]
        cnt_s[...] = jnp.zeros((_B, _T), jnp.float32)

    x = x_s[...]

    proj = jax.lax.dot_general(x, v_ref[0], (((1,), (1,)), ((), ())))

    mm = (jax.lax.broadcasted_iota(jnp.int32, (_TR, _T), 0) // _R
          == jax.lax.broadcasted_iota(jnp.int32, (_TR, _T), 1)
          ).astype(jnp.float32)
    logits = jax.lax.dot_general(proj * proj, mm, (((1,), (0,)), ((), ())),
                              precision=jax.lax.Precision.HIGHEST)
    onehot = _first_max_onehot(logits)
    cnt_s[...] += onehot

    mask = jax.lax.dot_general(onehot, mm, (((1,), (1,)), ((), ())),
                              precision=jax.lax.Precision.HIGHEST)
    mproj = proj * mask
    base = jax.lax.dot_general(x, w0_ref[0], (((1,), (1,)), ((), ())))
    delta = jax.lax.dot_general(mproj, u_ref[0], (((1,), (0,)), ((), ())))
    y = base + delta

    @pl.when(l < _L - 1)
    def _next():
        x_s[...] = jax.nn.gelu(y)

    @pl.when(l == _L - 1)
    def _head():
        maj = _first_max_onehot(cnt_s[...])
        hb = hb_ref[...]
        acc = jnp.zeros((_B, _C), jnp.float32)
        for t in range(_T):
            o_t = jax.lax.dot_general(y, hw_ref[t], (((1,), (1,)), ((), ())))
            acc += maj[:, t:t + 1] * (o_t + hb[t:t + 1, :])
        out_ref[...] = acc


def kernel(batch, W0, V, U, heads_W, heads_b):
    V2 = V.reshape(_L, _TR, _D)
    U2 = U.reshape(_L, _TR, _D)
    return pl.pallas_call(
        _smile_kernel,
        grid=(_L,),
        in_specs=[
            pl.BlockSpec((_B, _D), lambda l: (0, 0)),
            pl.BlockSpec((1, _D, _D), lambda l: (l, 0, 0)),
            pl.BlockSpec((1, _TR, _D), lambda l: (l, 0, 0)),
            pl.BlockSpec((1, _TR, _D), lambda l: (l, 0, 0)),
            pl.BlockSpec((_T, _C, _D), lambda l: (0, 0, 0)),
            pl.BlockSpec((_T, _C), lambda l: (0, 0)),
        ],
        out_specs=pl.BlockSpec((_B, _C), lambda l: (0, 0)),
        out_shape=jax.ShapeDtypeStruct((_B, _C), jnp.float32),
        scratch_shapes=[
            pltpu.VMEM((_B, _D), jnp.float32),
            pltpu.VMEM((_B, _T), jnp.float32),
        ],
    )(batch, W0, V2, U2, heads_W, heads_b)


# speedup vs baseline: 1.3706x; 1.3706x over previous
import jax
import jax.numpy as jnp
from jax.experimental import pallas as pl
from jax.experimental.pallas import tpu as pltpu

_L, _B, _D, _T, _R, _C = 12, 1024, 768, 8, 16, 100
_TR = _T * _R
_LP = 2
_NS = _L // _LP


def _first_max_onehot(scores):
    t = scores.shape[-1]
    m = jnp.max(scores, axis=-1, keepdims=True)
    is_max = (scores == m).astype(jnp.float32)
    tri = (jax.lax.broadcasted_iota(jnp.int32, (t, t), 0)
           <= jax.lax.broadcasted_iota(jnp.int32, (t, t), 1)).astype(jnp.float32)
    csum = jax.lax.dot_general(is_max, tri, (((1,), (0,)), ((), ())),
                               precision=jax.lax.Precision.HIGHEST)
    return is_max * (csum == 1.0).astype(jnp.float32)


def _smile_kernel(x0_ref, w0_ref, v_ref, u_ref, hw_ref, hb_ref, out_ref,
                  x_s, cnt_s):
    l = pl.program_id(0)

    @pl.when(l == 0)
    def _init():
        x_s[...] = x0_ref[...]
        cnt_s[...] = jnp.zeros((_B, _T), jnp.float32)

    mm = (jax.lax.broadcasted_iota(jnp.int32, (_TR, _T), 0) // _R
          == jax.lax.broadcasted_iota(jnp.int32, (_TR, _T), 1)
          ).astype(jnp.float32)

    def _layer(x, j):
        proj = jax.lax.dot_general(x, v_ref[j], (((1,), (1,)), ((), ())))
        logits = jax.lax.dot_general(proj * proj, mm, (((1,), (0,)), ((), ())),
                                     precision=jax.lax.Precision.HIGHEST)
        onehot = _first_max_onehot(logits)
        cnt_s[...] += onehot
        mask = jax.lax.dot_general(onehot, mm, (((1,), (1,)), ((), ())),
                                   precision=jax.lax.Precision.HIGHEST)
        mproj = proj * mask
        base = jax.lax.dot_general(x, w0_ref[j], (((1,), (1,)), ((), ())))
        delta = jax.lax.dot_general(mproj, u_ref[j], (((1,), (0,)), ((), ())))
        return base + delta

    y0 = _layer(x_s[...], 0)
    y1 = _layer(jax.nn.gelu(y0), 1)

    @pl.when(l < _NS - 1)
    def _next():
        x_s[...] = jax.nn.gelu(y1)

    @pl.when(l == _NS - 1)
    def _head():
        maj = _first_max_onehot(cnt_s[...])
        hb = hb_ref[...]
        acc = jnp.zeros((_B, _C), jnp.float32)
        for t in range(_T):
            o_t = jax.lax.dot_general(y1, hw_ref[t], (((1,), (1,)), ((), ())))
            acc += maj[:, t:t + 1] * (o_t + hb[t:t + 1, :])
        out_ref[...] = acc


def kernel(batch, W0, V, U, heads_W, heads_b):
    V2 = V.reshape(_L, _TR, _D)
    U2 = U.transpose(0, 1, 3, 2).reshape(_L, _TR, _D)
    return pl.pallas_call(
        _smile_kernel,
        grid=(_NS,),
        in_specs=[
            pl.BlockSpec((_B, _D), lambda l: (0, 0)),
            pl.BlockSpec((_LP, _D, _D), lambda l: (l, 0, 0)),
            pl.BlockSpec((_LP, _TR, _D), lambda l: (l, 0, 0)),
            pl.BlockSpec((_LP, _TR, _D), lambda l: (l, 0, 0)),
            pl.BlockSpec((_T, _C, _D), lambda l: (0, 0, 0)),
            pl.BlockSpec((_T, _C), lambda l: (0, 0)),
        ],
        out_specs=pl.BlockSpec((_B, _C), lambda l: (0, 0)),
        out_shape=jax.ShapeDtypeStruct((_B, _C), jnp.float32),
        scratch_shapes=[
            pltpu.VMEM((_B, _D), jnp.float32),
            pltpu.VMEM((_B, _T), jnp.float32),
        ],
    )(batch, W0, V2, U2, heads_W, heads_b)


# speedup vs baseline: 1.5368x; 1.1212x over previous
import jax
import jax.numpy as jnp
from jax.experimental import pallas as pl
from jax.experimental.pallas import tpu as pltpu

_L, _B, _D, _T, _R, _C = 12, 1024, 768, 8, 16, 100
_TR = _T * _R
_LP = 1
_NS = _L // _LP


def _first_max_onehot(scores):
    t = scores.shape[-1]
    m = jnp.max(scores, axis=-1, keepdims=True)
    is_max = (scores == m).astype(jnp.float32)
    tri = (jax.lax.broadcasted_iota(jnp.int32, (t, t), 0)
           <= jax.lax.broadcasted_iota(jnp.int32, (t, t), 1)).astype(jnp.float32)
    csum = jax.lax.dot_general(is_max, tri, (((1,), (0,)), ((), ())),
                               precision=jax.lax.Precision.HIGHEST)
    return is_max * (csum == 1.0).astype(jnp.float32)


def _smile_kernel(x0_ref, w0_ref, v_ref, u_ref, hw_ref, hb_ref, out_ref,
                  x_s, cnt_s):
    l = pl.program_id(0)

    @pl.when(l == 0)
    def _init():
        x_s[...] = x0_ref[...]
        cnt_s[...] = jnp.zeros((_B, _T), jnp.float32)

    mm = (jax.lax.broadcasted_iota(jnp.int32, (_TR, _T), 0) // _R
          == jax.lax.broadcasted_iota(jnp.int32, (_TR, _T), 1)
          ).astype(jnp.float32)

    def _layer(x, j):
        proj = jax.lax.dot_general(x, v_ref[j], (((1,), (1,)), ((), ())))
        logits = jax.lax.dot_general(proj * proj, mm, (((1,), (0,)), ((), ())),
                                     precision=jax.lax.Precision.HIGHEST)
        onehot = _first_max_onehot(logits)
        cnt_s[...] += onehot
        mask = jax.lax.dot_general(onehot, mm, (((1,), (1,)), ((), ())),
                                   precision=jax.lax.Precision.HIGHEST)
        mproj = proj * mask
        base = jax.lax.dot_general(x, w0_ref[j], (((1,), (1,)), ((), ())))
        delta = jax.lax.dot_general(mproj, u_ref[j], (((1,), (0,)), ((), ())))
        return base + delta

    y1 = _layer(x_s[...], 0)

    @pl.when(l < _NS - 1)
    def _next():
        x_s[...] = jax.nn.gelu(y1)

    @pl.when(l == _NS - 1)
    def _head():
        maj = _first_max_onehot(cnt_s[...])
        hb = hb_ref[...]
        acc = jnp.zeros((_B, _C), jnp.float32)
        for t in range(_T):
            o_t = jax.lax.dot_general(y1, hw_ref[t], (((1,), (1,)), ((), ())))
            acc += maj[:, t:t + 1] * (o_t + hb[t:t + 1, :])
        out_ref[...] = acc


def kernel(batch, W0, V, U, heads_W, heads_b):
    V2 = V.reshape(_L, _TR, _D)
    U2 = U.transpose(0, 1, 3, 2).reshape(_L, _TR, _D)
    return pl.pallas_call(
        _smile_kernel,
        grid=(_NS,),
        in_specs=[
            pl.BlockSpec((_B, _D), lambda l: (0, 0)),
            pl.BlockSpec((_LP, _D, _D), lambda l: (l, 0, 0)),
            pl.BlockSpec((_LP, _TR, _D), lambda l: (l, 0, 0)),
            pl.BlockSpec((_LP, _TR, _D), lambda l: (l, 0, 0)),
            pl.BlockSpec((_T, _C, _D), lambda l: (0, 0, 0)),
            pl.BlockSpec((_T, _C), lambda l: (0, 0)),
        ],
        out_specs=pl.BlockSpec((_B, _C), lambda l: (0, 0)),
        out_shape=jax.ShapeDtypeStruct((_B, _C), jnp.float32),
        scratch_shapes=[
            pltpu.VMEM((_B, _D), jnp.float32),
            pltpu.VMEM((_B, _T), jnp.float32),
        ],
    )(batch, W0, V2, U2, heads_W, heads_b)


# speedup vs baseline: 1.8938x; 1.2323x over previous
import jax
import jax.numpy as jnp
from jax.experimental import pallas as pl
from jax.experimental.pallas import tpu as pltpu

_L, _B, _D, _T, _R, _C = 12, 1024, 768, 8, 16, 100
_TR = _T * _R
_LP = 1
_NS = _L // _LP


def _first_max_onehot(scores):
    t = scores.shape[-1]
    m = jnp.max(scores, axis=-1, keepdims=True)
    is_max = (scores == m).astype(jnp.float32)
    tri = (jax.lax.broadcasted_iota(jnp.int32, (t, t), 0)
           <= jax.lax.broadcasted_iota(jnp.int32, (t, t), 1)).astype(jnp.float32)
    csum = jax.lax.dot_general(is_max, tri, (((1,), (0,)), ((), ())))
    return is_max * (csum == 1.0).astype(jnp.float32)


def _smile_kernel(x0_ref, w0_ref, v_ref, u_ref, hw_ref, hb_ref, out_ref,
                  x_s, cnt_s):
    l = pl.program_id(0)

    @pl.when(l == 0)
    def _init():
        x_s[...] = x0_ref[...]
        cnt_s[...] = jnp.zeros((_B, _T), jnp.float32)

    mm = (jax.lax.broadcasted_iota(jnp.int32, (_TR, _T), 0) // _R
          == jax.lax.broadcasted_iota(jnp.int32, (_TR, _T), 1)
          ).astype(jnp.float32)

    def _layer(x, j):
        proj = jax.lax.dot_general(x, v_ref[j], (((1,), (1,)), ((), ())))
        logits = jax.lax.dot_general(proj * proj, mm, (((1,), (0,)), ((), ())),
                                     precision=jax.lax.Precision.HIGHEST)
        onehot = _first_max_onehot(logits)
        cnt_s[...] += onehot
        mask = jax.lax.dot_general(onehot, mm, (((1,), (1,)), ((), ())))
        mproj = proj * mask
        base = jax.lax.dot_general(x, w0_ref[j], (((1,), (1,)), ((), ())))
        delta = jax.lax.dot_general(mproj, u_ref[j], (((1,), (0,)), ((), ())))
        return base + delta

    y1 = _layer(x_s[...], 0)

    @pl.when(l < _NS - 1)
    def _next():
        x_s[...] = jax.nn.gelu(y1)

    @pl.when(l == _NS - 1)
    def _head():
        maj = _first_max_onehot(cnt_s[...])
        hb = hb_ref[...]
        acc = jnp.zeros((_B, _C), jnp.float32)
        for t in range(_T):
            o_t = jax.lax.dot_general(y1, hw_ref[t], (((1,), (1,)), ((), ())))
            acc += maj[:, t:t + 1] * (o_t + hb[t:t + 1, :])
        out_ref[...] = acc


def kernel(batch, W0, V, U, heads_W, heads_b):
    V2 = V.reshape(_L, _TR, _D)
    U2 = U.transpose(0, 1, 3, 2).reshape(_L, _TR, _D)
    return pl.pallas_call(
        _smile_kernel,
        grid=(_NS,),
        in_specs=[
            pl.BlockSpec((_B, _D), lambda l: (0, 0)),
            pl.BlockSpec((_LP, _D, _D), lambda l: (l, 0, 0)),
            pl.BlockSpec((_LP, _TR, _D), lambda l: (l, 0, 0)),
            pl.BlockSpec((_LP, _TR, _D), lambda l: (l, 0, 0)),
            pl.BlockSpec((_T, _C, _D), lambda l: (0, 0, 0)),
            pl.BlockSpec((_T, _C), lambda l: (0, 0)),
        ],
        out_specs=pl.BlockSpec((_B, _C), lambda l: (0, 0)),
        out_shape=jax.ShapeDtypeStruct((_B, _C), jnp.float32),
        scratch_shapes=[
            pltpu.VMEM((_B, _D), jnp.float32),
            pltpu.VMEM((_B, _T), jnp.float32),
        ],
    )(batch, W0, V2, U2, heads_W, heads_b)


# speedup vs baseline: 2.1174x; 1.1180x over previous
import jax
import jax.numpy as jnp
from jax.experimental import pallas as pl
from jax.experimental.pallas import tpu as pltpu

_L, _B, _D, _T, _R, _C = 12, 1024, 768, 8, 16, 100
_TR = _T * _R
_LP = 1
_NS = _L // _LP


def _first_max_onehot(scores):
    t = scores.shape[-1]
    m = jnp.max(scores, axis=-1, keepdims=True)
    is_max = (scores == m).astype(jnp.float32)
    tri = (jax.lax.broadcasted_iota(jnp.int32, (t, t), 0)
           <= jax.lax.broadcasted_iota(jnp.int32, (t, t), 1)).astype(jnp.float32)
    csum = jax.lax.dot_general(is_max, tri, (((1,), (0,)), ((), ())))
    return is_max * (csum == 1.0).astype(jnp.float32)


def _smile_kernel(x0_ref, w0_ref, vh_ref, vn_ref, u_ref, hw_ref, hb_ref,
                  out_ref, x_s, cnt_s, proj_s, log_s):
    l = pl.program_id(0)

    mm = (jax.lax.broadcasted_iota(jnp.int32, (_TR, _T), 0) // _R
          == jax.lax.broadcasted_iota(jnp.int32, (_TR, _T), 1)
          ).astype(jnp.float32)

    def _route(x, v):
        proj = jax.lax.dot_general(x, v, (((1,), (1,)), ((), ())))
        logits = jax.lax.dot_general(proj * proj, mm, (((1,), (0,)), ((), ())),
                                     precision=jax.lax.Precision.HIGHEST)
        return proj, logits

    @pl.when(l == 0)
    def _init():
        x0 = x0_ref[...]
        x_s[...] = x0
        cnt_s[...] = jnp.zeros((_B, _T), jnp.float32)
        proj_s[...], log_s[...] = _route(x0, vh_ref[0])

    x = x_s[...]
    proj = proj_s[...]
    onehot = _first_max_onehot(log_s[...])
    cnt_s[...] += onehot
    mask = jax.lax.dot_general(onehot, mm, (((1,), (1,)), ((), ())))
    mproj = proj * mask
    base = jax.lax.dot_general(x, w0_ref[0], (((1,), (1,)), ((), ())))
    delta = jax.lax.dot_general(mproj, u_ref[0], (((1,), (0,)), ((), ())))
    y1 = base + delta

    @pl.when(l < _NS - 1)
    def _next():
        xg = jax.nn.gelu(y1)
        x_s[...] = xg
        proj_s[...], log_s[...] = _route(xg, vn_ref[0])

    @pl.when(l == _NS - 1)
    def _head():
        maj = _first_max_onehot(cnt_s[...])
        hb = hb_ref[...]
        acc = jnp.zeros((_B, _C), jnp.float32)
        for t in range(_T):
            o_t = jax.lax.dot_general(y1, hw_ref[t], (((1,), (1,)), ((), ())))
            acc += maj[:, t:t + 1] * (o_t + hb[t:t + 1, :])
        out_ref[...] = acc


def kernel(batch, W0, V, U, heads_W, heads_b):
    V2 = V.reshape(_L, _TR, _D)
    U2 = U.transpose(0, 1, 3, 2).reshape(_L, _TR, _D)
    return pl.pallas_call(
        _smile_kernel,
        grid=(_NS,),
        in_specs=[
            pl.BlockSpec((_B, _D), lambda l: (0, 0)),
            pl.BlockSpec((_LP, _D, _D), lambda l: (l, 0, 0)),
            pl.BlockSpec((_LP, _TR, _D), lambda l: (0, 0, 0)),
            pl.BlockSpec((_LP, _TR, _D),
                         lambda l: (jnp.minimum(l + 1, _NS - 1), 0, 0)),
            pl.BlockSpec((_LP, _TR, _D), lambda l: (l, 0, 0)),
            pl.BlockSpec((_T, _C, _D), lambda l: (0, 0, 0)),
            pl.BlockSpec((_T, _C), lambda l: (0, 0)),
        ],
        out_specs=pl.BlockSpec((_B, _C), lambda l: (0, 0)),
        out_shape=jax.ShapeDtypeStruct((_B, _C), jnp.float32),
        scratch_shapes=[
            pltpu.VMEM((_B, _D), jnp.float32),
            pltpu.VMEM((_B, _T), jnp.float32),
            pltpu.VMEM((_B, _TR), jnp.float32),
            pltpu.VMEM((_B, _T), jnp.float32),
        ],
    )(batch, W0, V2, V2, U2, heads_W, heads_b)
